# baseline (speedup 1.0000x reference)
import jax
import jax.numpy as jnp
from jax.experimental import pallas as pl

N = 10000
E = 320000
D = 128
H = 128
G = 64
EPS = 1e-5
NPAD = 10016


def _dense_body(agg_ref, w_ref, b_ref, g_ref, bt_ref, o_ref):
    agg = agg_ref[...]
    h = jnp.dot(agg, w_ref[...], preferred_element_type=jnp.float32) + b_ref[...]
    row = jax.lax.broadcasted_iota(jnp.int32, (NPAD, 1), 0)
    mask = (row < N).astype(jnp.float32)
    mean = jnp.sum(h * mask, axis=0, keepdims=True) * (1.0 / N)
    d = (h - mean) * mask
    var = jnp.sum(d * d, axis=0, keepdims=True) * (1.0 / N)
    hn = (h - mean) * jax.lax.rsqrt(var + EPS) * g_ref[...] + bt_ref[...]
    o_ref[...] = jnp.maximum(hn, 0.0) * mask


def _dense_layer(agg, W, b, g, bt):
    K = agg.shape[1]
    return pl.pallas_call(
        _dense_body,
        out_shape=jax.ShapeDtypeStruct((NPAD, H), jnp.float32),
    )(agg, W, b.reshape(1, H), g.reshape(1, H), bt.reshape(1, H))


def kernel(x, edge_index, edge_attr, batch, W0, b0, g0, bt0, W1, b1, g1, bt1, Wout, bout):
    src = edge_index[0]
    dst = edge_index[1]

    def aggregate(feats):
        m = jnp.concatenate([feats[src], edge_attr], axis=1)
        agg = jax.ops.segment_max(m, dst, num_segments=N)
        agg = jnp.where(jnp.isfinite(agg), agg, 0.0)
        return jnp.pad(agg, ((0, NPAD - N), (0, 0)))

    h0 = _dense_layer(aggregate(x), W0, b0, g0, bt0)
    h1 = _dense_layer(aggregate(h0[:N]), W1, b1, g1, bt1)

    pooled = jax.ops.segment_max(h1[:N], batch, num_segments=G)
    pooled = jnp.where(jnp.isfinite(pooled), pooled, 0.0)
    return pooled @ Wout + bout
